# baseline (speedup 1.0000x reference)
import functools

import jax
import jax.numpy as jnp
from jax import lax
from jax.experimental import pallas as pl
from jax.experimental.pallas import tpu as pltpu
from jax.experimental.pallas import tpu_sc as plsc

_PACK = 4
_ROW = 128


def kernel(user_id, item_id, diff, user_table, item_table, W_out, b_out):
    B = user_id.shape[0]
    D = user_table.shape[1]
    V = user_table.shape[0]
    info = plsc.get_sparse_core_info()
    NC, NS, L = info.num_cores, info.num_subcores, info.num_lanes
    NW = NC * NS
    b_per_w = B // NW
    CH = 128
    n_ch = b_per_w // CH
    HALF = b_per_w // 2
    n_grp = HALF // L

    ut2 = user_table.reshape(V // _PACK, _ROW)
    it2 = item_table.reshape(V // _PACK, _ROW)
    uhi3 = (user_id // _PACK).reshape(NW, n_ch, CH)
    ihi3 = (item_id // _PACK).reshape(NW, n_ch, CH)
    uoff2 = ((user_id % _PACK) * D).reshape(NW, b_per_w)
    ioff2 = ((item_id % _PACK) * D).reshape(NW, b_per_w)
    w16 = jnp.broadcast_to(W_out.reshape(1), (L,))
    b16 = jnp.broadcast_to(b_out, (L,))

    mesh = plsc.VectorSubcoreMesh(core_axis_name="c", subcore_axis_name="s")

    @functools.partial(
        pl.kernel,
        mesh=mesh,
        out_type=jax.ShapeDtypeStruct((B,), jnp.float32),
        scratch_types=[
            pltpu.VMEM((n_ch, CH), jnp.int32),
            pltpu.VMEM((n_ch, CH), jnp.int32),
            pltpu.VMEM((b_per_w,), jnp.int32),
            pltpu.VMEM((b_per_w,), jnp.int32),
            pltpu.VMEM((HALF, _ROW), jnp.float32),
            pltpu.VMEM((HALF, _ROW), jnp.float32),
            pltpu.VMEM((b_per_w,), jnp.float32),
            pltpu.VMEM((b_per_w,), jnp.float32),
            pltpu.VMEM((L,), jnp.float32),
            pltpu.VMEM((L,), jnp.float32),
            pltpu.SemaphoreType.DMA,
        ],
        compiler_params=pltpu.CompilerParams(needs_layout_passes=False),
    )
    def mf_kernel(uhi_h, ihi_h, uoff_h, ioff_h, diff_h, ut_h, it_h, w_h, b_h,
                  out_h, uhi_v, ihi_v, uoff_v, ioff_v, urows_v, irows_v,
                  diff_v, out_v, w_v, b_v, sem):
        wid = lax.axis_index("s") * NC + lax.axis_index("c")
        base = wid * b_per_w

        pltpu.sync_copy(uhi_h.at[wid], uhi_v)
        pltpu.sync_copy(ihi_h.at[wid], ihi_v)
        pltpu.sync_copy(uoff_h.at[wid], uoff_v)
        pltpu.sync_copy(ioff_h.at[wid], ioff_v)
        pltpu.sync_copy(diff_h.at[pl.ds(base, b_per_w)], diff_v)
        pltpu.sync_copy(w_h, w_v)
        pltpu.sync_copy(b_h, b_v)

        w = w_v[...]
        b = b_v[...]
        riota = lax.iota(jnp.int32, L)

        for p in range(b_per_w // HALF):
            copies = []
            for jj in range(HALF // CH):
                j = p * (HALF // CH) + jj
                copies.append(pltpu.async_copy(
                    ut_h.at[uhi_v.at[j]],
                    urows_v.at[pl.ds(jj * CH, CH)], sem))
                copies.append(pltpu.async_copy(
                    it_h.at[ihi_v.at[j]],
                    irows_v.at[pl.ds(jj * CH, CH)], sem))
            for c in copies:
                c.wait()

            def body(g, carry, p=p):
                lrows = riota + g * L
                gbase = p * HALF + g * L
                uoff = uoff_v[pl.ds(gbase, L)]
                ioff = ioff_v[pl.ds(gbase, L)]
                acc = jnp.zeros((L,), jnp.float32)
                for d in range(D):
                    uc = plsc.load_gather(urows_v, [lrows, uoff + d])
                    ic = plsc.load_gather(irows_v, [lrows, ioff + d])
                    acc = acc + uc * ic
                sig = 1.0 / (1.0 + jnp.exp(-acc))
                dv = diff_v[pl.ds(gbase, L)]
                y = sig * dv * w + b
                out_v[pl.ds(gbase, L)] = 1.0 / (1.0 + jnp.exp(-y))
                return carry

            lax.fori_loop(0, n_grp, body, 0)

        pltpu.sync_copy(out_v, out_h.at[pl.ds(base, b_per_w)])

    return mf_kernel(uhi3, ihi3, uoff2, ioff2, diff, ut2, it2, w16, b16)


# speedup vs baseline: 2.6051x; 2.6051x over previous
import functools

import jax
import jax.numpy as jnp
from jax import lax
from jax.experimental import pallas as pl
from jax.experimental.pallas import tpu as pltpu
from jax.experimental.pallas import tpu_sc as plsc

_LANES = 128


def kernel(user_id, item_id, diff, user_table, item_table, W_out, b_out):
    B = user_id.shape[0]
    D = user_table.shape[1]
    V = user_table.shape[0]
    info = plsc.get_sparse_core_info()
    NC, NS, L = info.num_cores, info.num_subcores, info.num_lanes
    NW = NC * NS
    b_per_w = B // NW
    HALF = b_per_w // 2
    n_grp = HALF // L
    n_pipe = HALF // 2
    NDB = D // 8

    utT = user_table.T.reshape(NDB, 8, V)
    itT = item_table.T.reshape(NDB, 8, V)
    dblk_ix = jnp.arange(NDB, dtype=jnp.int32)
    w16 = jnp.broadcast_to(W_out.reshape(1), (L,))
    b16 = jnp.broadcast_to(b_out, (L,))

    mesh = plsc.VectorSubcoreMesh(core_axis_name="c", subcore_axis_name="s")

    @functools.partial(
        pl.kernel,
        mesh=mesh,
        out_type=jax.ShapeDtypeStruct((B,), jnp.float32),
        scratch_types=[
            pltpu.VMEM((b_per_w + L,), jnp.int32),
            pltpu.VMEM((b_per_w + L,), jnp.int32),
            pltpu.VMEM((NDB,), jnp.int32),
            pltpu.VMEM((NDB, 8, _LANES), jnp.float32),
            pltpu.VMEM((NDB, 8, _LANES), jnp.float32),
            pltpu.VMEM((NDB, 8, _LANES), jnp.float32),
            pltpu.VMEM((NDB, 8, _LANES), jnp.float32),
            pltpu.VMEM((HALF, D), jnp.float32),
            pltpu.VMEM((HALF, D), jnp.float32),
            pltpu.VMEM((b_per_w,), jnp.float32),
            pltpu.VMEM((b_per_w,), jnp.float32),
            pltpu.VMEM((L,), jnp.float32),
            pltpu.VMEM((L,), jnp.float32),
            pltpu.SemaphoreType.DMA,
            pltpu.SemaphoreType.DMA,
        ],
        compiler_params=pltpu.CompilerParams(needs_layout_passes=False),
    )
    def mf_kernel(uid_h, iid_h, diff_h, ut_h, it_h, w_h, b_h, dix_h, out_h,
                  uid_v, iid_v, dix_v, ua_v, ia_v, ub_v, ib_v, urow_v,
                  irow_v, diff_v, out_v, w_v, b_v, semA, semB):
        wid = lax.axis_index("s") * NC + lax.axis_index("c")
        base = wid * b_per_w

        pltpu.sync_copy(uid_h.at[pl.ds(base, b_per_w)],
                        uid_v.at[pl.ds(0, b_per_w)])
        pltpu.sync_copy(iid_h.at[pl.ds(base, b_per_w)],
                        iid_v.at[pl.ds(0, b_per_w)])
        pltpu.sync_copy(diff_h.at[pl.ds(base, b_per_w)], diff_v)
        pltpu.sync_copy(w_h, w_v)
        pltpu.sync_copy(b_h, b_v)
        pltpu.sync_copy(dix_h, dix_v)

        w = w_v[...]
        b = b_v[...]
        riota = lax.iota(jnp.int32, L)
        diota = lax.iota(jnp.int32, L)
        db_lo = lax.shift_right_logical(diota, 3)
        db_hi = lax.shift_right_logical(diota + L, 3)
        di_sub = diota & 7

        def fire(uid, iid, bu, bi, sem):
            ub0 = pl.multiple_of(uid & ~(_LANES - 1), _LANES)
            pltpu.async_copy(
                ut_h.at[dix_v, :, pl.ds(ub0, _LANES)], bu, sem)
            ib0 = pl.multiple_of(iid & ~(_LANES - 1), _LANES)
            pltpu.async_copy(
                it_h.at[dix_v, :, pl.ds(ib0, _LANES)], bi, sem)

        def drain(bu, bi, sem):
            pltpu.make_async_copy(
                ut_h.at[dix_v, :, pl.ds(0, _LANES)], bu, sem).wait()
            pltpu.make_async_copy(
                it_h.at[dix_v, :, pl.ds(0, _LANES)], bi, sem).wait()

        def extract(uid, iid, m, bu, bi):
            ulane = jnp.full((L,), uid & (_LANES - 1), jnp.int32)
            urow_v[m, pl.ds(0, L)] = plsc.load_gather(
                bu, [db_lo, di_sub, ulane])
            urow_v[m, pl.ds(L, L)] = plsc.load_gather(
                bu, [db_hi, di_sub, ulane])
            ilane = jnp.full((L,), iid & (_LANES - 1), jnp.int32)
            irow_v[m, pl.ds(0, L)] = plsc.load_gather(
                bi, [db_lo, di_sub, ilane])
            irow_v[m, pl.ds(L, L)] = plsc.load_gather(
                bi, [db_hi, di_sub, ilane])

        for half in range(2):
            h0 = half * HALF
            uv0 = uid_v[pl.ds(h0, L)]
            iv0 = iid_v[pl.ds(h0, L)]
            fire(uv0[0], iv0[0], ua_v, ia_v, semA)

            def pipe_body(j, carry, h0=h0):
                loc0 = 2 * j
                n0 = h0 + loc0
                uv = uid_v[pl.ds(n0, L)]
                iv = iid_v[pl.ds(n0, L)]
                fire(uv[1], iv[1], ub_v, ib_v, semB)
                drain(ua_v, ia_v, semA)
                extract(uv[0], iv[0], loc0, ua_v, ia_v)

                @pl.when(j < n_pipe - 1)
                def _():
                    fire(uv[2], iv[2], ua_v, ia_v, semA)

                drain(ub_v, ib_v, semB)
                extract(uv[1], iv[1], loc0 + 1, ub_v, ib_v)
                return carry

            lax.fori_loop(0, n_pipe, pipe_body, 0)

            def dot_body(g, carry, h0=h0):
                rows = riota + g * L
                acc = jnp.zeros((L,), jnp.float32)
                for d in range(D):
                    dcol = jnp.full((L,), d, jnp.int32)
                    uc = plsc.load_gather(urow_v, [rows, dcol])
                    ic = plsc.load_gather(irow_v, [rows, dcol])
                    acc = acc + uc * ic
                sig = 1.0 / (1.0 + jnp.exp(-acc))
                dv = diff_v[pl.ds(h0 + g * L, L)]
                y = sig * dv * w + b
                out_v[pl.ds(h0 + g * L, L)] = 1.0 / (1.0 + jnp.exp(-y))
                return carry

            lax.fori_loop(0, n_grp, dot_body, 0)

        pltpu.sync_copy(out_v, out_h.at[pl.ds(base, b_per_w)])

    return mf_kernel(user_id, item_id, diff, utT, itT, w16, b16, dblk_ix)


# speedup vs baseline: 3.1160x; 1.1961x over previous
import functools

import jax
import jax.numpy as jnp
from jax import lax
from jax.experimental import pallas as pl
from jax.experimental.pallas import tpu as pltpu
from jax.experimental.pallas import tpu_sc as plsc

_LANES = 128


def kernel(user_id, item_id, diff, user_table, item_table, W_out, b_out):
    B = user_id.shape[0]
    D = user_table.shape[1]
    V = user_table.shape[0]
    info = plsc.get_sparse_core_info()
    NC, NS, L = info.num_cores, info.num_subcores, info.num_lanes
    NW = NC * NS
    b_per_w = B // NW
    HALF = b_per_w // 2
    n_grp = HALF // L
    n_pipe = HALF // 4
    NDB = D // 8

    utT = user_table.T.reshape(NDB, 8, V)
    itT = item_table.T.reshape(NDB, 8, V)
    dblk_ix = jnp.arange(NDB, dtype=jnp.int32)
    w16 = jnp.broadcast_to(W_out.reshape(1), (L,))
    b16 = jnp.broadcast_to(b_out, (L,))

    mesh = plsc.VectorSubcoreMesh(core_axis_name="c", subcore_axis_name="s")

    @functools.partial(
        pl.kernel,
        mesh=mesh,
        out_type=jax.ShapeDtypeStruct((B,), jnp.float32),
        scratch_types=[
            pltpu.VMEM((b_per_w + L,), jnp.int32),
            pltpu.VMEM((b_per_w + L,), jnp.int32),
            pltpu.VMEM((NDB,), jnp.int32),
            pltpu.VMEM((2, NDB, 8, _LANES), jnp.float32),
            pltpu.VMEM((2, NDB, 8, _LANES), jnp.float32),
            pltpu.VMEM((2, NDB, 8, _LANES), jnp.float32),
            pltpu.VMEM((2, NDB, 8, _LANES), jnp.float32),
            pltpu.VMEM((HALF, D), jnp.float32),
            pltpu.VMEM((HALF, D), jnp.float32),
            pltpu.VMEM((b_per_w,), jnp.float32),
            pltpu.VMEM((b_per_w,), jnp.float32),
            pltpu.VMEM((L,), jnp.float32),
            pltpu.VMEM((L,), jnp.float32),
            pltpu.SemaphoreType.DMA,
            pltpu.SemaphoreType.DMA,
        ],
        compiler_params=pltpu.CompilerParams(needs_layout_passes=False),
    )
    def mf_kernel(uid_h, iid_h, diff_h, ut_h, it_h, w_h, b_h, dix_h, out_h,
                  uid_v, iid_v, dix_v, ua_v, ia_v, ub_v, ib_v, urow_v,
                  irow_v, diff_v, out_v, w_v, b_v, semA, semB):
        wid = lax.axis_index("s") * NC + lax.axis_index("c")
        base = wid * b_per_w

        pltpu.sync_copy(uid_h.at[pl.ds(base, b_per_w)],
                        uid_v.at[pl.ds(0, b_per_w)])
        pltpu.sync_copy(iid_h.at[pl.ds(base, b_per_w)],
                        iid_v.at[pl.ds(0, b_per_w)])
        pltpu.sync_copy(diff_h.at[pl.ds(base, b_per_w)], diff_v)
        pltpu.sync_copy(w_h, w_v)
        pltpu.sync_copy(b_h, b_v)
        pltpu.sync_copy(dix_h, dix_v)

        w = w_v[...]
        b = b_v[...]
        riota = lax.iota(jnp.int32, L)
        diota = lax.iota(jnp.int32, L)
        db_lo = lax.shift_right_logical(diota, 3)
        db_hi = lax.shift_right_logical(diota + L, 3)
        di_sub = diota & 7

        def fire(uid0, iid0, uid1, iid1, bu, bi, sem):
            for s, (uid, iid) in enumerate(((uid0, iid0), (uid1, iid1))):
                ub0 = pl.multiple_of(uid & ~(_LANES - 1), _LANES)
                pltpu.async_copy(
                    ut_h.at[dix_v, :, pl.ds(ub0, _LANES)], bu.at[s], sem)
                ib0 = pl.multiple_of(iid & ~(_LANES - 1), _LANES)
                pltpu.async_copy(
                    it_h.at[dix_v, :, pl.ds(ib0, _LANES)], bi.at[s], sem)

        def drain(bu, bi, sem):
            for s in range(2):
                pltpu.make_async_copy(
                    ut_h.at[dix_v, :, pl.ds(0, _LANES)], bu.at[s], sem).wait()
                pltpu.make_async_copy(
                    it_h.at[dix_v, :, pl.ds(0, _LANES)], bi.at[s], sem).wait()

        def extract(uid, iid, s, m, bu, bi):
            sfull = jnp.full((L,), s, jnp.int32)
            ulane = jnp.full((L,), uid & (_LANES - 1), jnp.int32)
            urow_v[m, pl.ds(0, L)] = plsc.load_gather(
                bu, [sfull, db_lo, di_sub, ulane])
            urow_v[m, pl.ds(L, L)] = plsc.load_gather(
                bu, [sfull, db_hi, di_sub, ulane])
            ilane = jnp.full((L,), iid & (_LANES - 1), jnp.int32)
            irow_v[m, pl.ds(0, L)] = plsc.load_gather(
                bi, [sfull, db_lo, di_sub, ilane])
            irow_v[m, pl.ds(L, L)] = plsc.load_gather(
                bi, [sfull, db_hi, di_sub, ilane])

        for half in range(2):
            h0 = half * HALF
            uv0 = uid_v[pl.ds(h0, L)]
            iv0 = iid_v[pl.ds(h0, L)]
            fire(uv0[0], iv0[0], uv0[1], iv0[1], ua_v, ia_v, semA)

            def pipe_body(j, carry, h0=h0):
                loc0 = 4 * j
                n0 = h0 + loc0
                uv = uid_v[pl.ds(n0, L)]
                iv = iid_v[pl.ds(n0, L)]
                fire(uv[2], iv[2], uv[3], iv[3], ub_v, ib_v, semB)
                drain(ua_v, ia_v, semA)
                extract(uv[0], iv[0], 0, loc0, ua_v, ia_v)
                extract(uv[1], iv[1], 1, loc0 + 1, ua_v, ia_v)

                @pl.when(j < n_pipe - 1)
                def _():
                    fire(uv[4], iv[4], uv[5], iv[5], ua_v, ia_v, semA)

                drain(ub_v, ib_v, semB)
                extract(uv[2], iv[2], 0, loc0 + 2, ub_v, ib_v)
                extract(uv[3], iv[3], 1, loc0 + 3, ub_v, ib_v)
                return carry

            lax.fori_loop(0, n_pipe, pipe_body, 0)

            def dot_body(g, carry, h0=h0):
                rows = riota + g * L
                acc = jnp.zeros((L,), jnp.float32)
                for d in range(D):
                    dcol = jnp.full((L,), d, jnp.int32)
                    uc = plsc.load_gather(urow_v, [rows, dcol])
                    ic = plsc.load_gather(irow_v, [rows, dcol])
                    acc = acc + uc * ic
                sig = 1.0 / (1.0 + jnp.exp(-acc))
                dv = diff_v[pl.ds(h0 + g * L, L)]
                y = sig * dv * w + b
                out_v[pl.ds(h0 + g * L, L)] = 1.0 / (1.0 + jnp.exp(-y))
                return carry

            lax.fori_loop(0, n_grp, dot_body, 0)

        pltpu.sync_copy(out_v, out_h.at[pl.ds(base, b_per_w)])

    return mf_kernel(user_id, item_id, diff, utT, itT, w16, b16, dblk_ix)


# speedup vs baseline: 3.2023x; 1.0277x over previous
import functools

import jax
import jax.numpy as jnp
from jax import lax
from jax.experimental import pallas as pl
from jax.experimental.pallas import tpu as pltpu
from jax.experimental.pallas import tpu_sc as plsc

_LANES = 128
_NS = 3


def kernel(user_id, item_id, diff, user_table, item_table, W_out, b_out):
    B = user_id.shape[0]
    D = user_table.shape[1]
    V = user_table.shape[0]
    info = plsc.get_sparse_core_info()
    NC, NS, L = info.num_cores, info.num_subcores, info.num_lanes
    NW = NC * NS
    b_per_w = B // NW
    QTR = b_per_w // 4
    n_grp = QTR // L
    n_pipe = QTR // (2 * _NS)
    tail = QTR - 2 * _NS * n_pipe
    NDB = D // 8

    utT = user_table.T.reshape(NDB, 8, V)
    itT = item_table.T.reshape(NDB, 8, V)
    dblk_ix = jnp.arange(NDB, dtype=jnp.int32)
    w16 = jnp.broadcast_to(W_out.reshape(1), (L,))
    b16 = jnp.broadcast_to(b_out, (L,))

    mesh = plsc.VectorSubcoreMesh(core_axis_name="c", subcore_axis_name="s")

    @functools.partial(
        pl.kernel,
        mesh=mesh,
        out_type=jax.ShapeDtypeStruct((B,), jnp.float32),
        scratch_types=[
            pltpu.VMEM((b_per_w + L,), jnp.int32),
            pltpu.VMEM((b_per_w + L,), jnp.int32),
            pltpu.VMEM((NDB,), jnp.int32),
            pltpu.VMEM((_NS, NDB, 8, _LANES), jnp.float32),
            pltpu.VMEM((_NS, NDB, 8, _LANES), jnp.float32),
            pltpu.VMEM((_NS, NDB, 8, _LANES), jnp.float32),
            pltpu.VMEM((_NS, NDB, 8, _LANES), jnp.float32),
            pltpu.VMEM((QTR, D), jnp.float32),
            pltpu.VMEM((QTR, D), jnp.float32),
            pltpu.VMEM((b_per_w,), jnp.float32),
            pltpu.VMEM((b_per_w,), jnp.float32),
            pltpu.VMEM((L,), jnp.float32),
            pltpu.VMEM((L,), jnp.float32),
            pltpu.SemaphoreType.DMA,
            pltpu.SemaphoreType.DMA,
        ],
        compiler_params=pltpu.CompilerParams(needs_layout_passes=False),
    )
    def mf_kernel(uid_h, iid_h, diff_h, ut_h, it_h, w_h, b_h, dix_h, out_h,
                  uid_v, iid_v, dix_v, ua_v, ia_v, ub_v, ib_v, urow_v,
                  irow_v, diff_v, out_v, w_v, b_v, semA, semB):
        wid = lax.axis_index("s") * NC + lax.axis_index("c")
        base = wid * b_per_w

        pltpu.sync_copy(uid_h.at[pl.ds(base, b_per_w)],
                        uid_v.at[pl.ds(0, b_per_w)])
        pltpu.sync_copy(iid_h.at[pl.ds(base, b_per_w)],
                        iid_v.at[pl.ds(0, b_per_w)])
        pltpu.sync_copy(diff_h.at[pl.ds(base, b_per_w)], diff_v)
        pltpu.sync_copy(w_h, w_v)
        pltpu.sync_copy(b_h, b_v)
        pltpu.sync_copy(dix_h, dix_v)

        zpad = jnp.zeros((L,), jnp.int32)
        uid_v[pl.ds(b_per_w, L)] = zpad
        iid_v[pl.ds(b_per_w, L)] = zpad

        w = w_v[...]
        b = b_v[...]
        riota = lax.iota(jnp.int32, L)
        diota = lax.iota(jnp.int32, L)
        db_lo = lax.shift_right_logical(diota, 3)
        db_hi = lax.shift_right_logical(diota + L, 3)
        di_sub = diota & 7

        def fire(uv, iv, k0, bu, bi, sem):
            for s in range(_NS):
                uid = uv[k0 + s]
                ub0 = pl.multiple_of(uid & ~(_LANES - 1), _LANES)
                pltpu.async_copy(
                    ut_h.at[dix_v, :, pl.ds(ub0, _LANES)], bu.at[s], sem)
                iid = iv[k0 + s]
                ib0 = pl.multiple_of(iid & ~(_LANES - 1), _LANES)
                pltpu.async_copy(
                    it_h.at[dix_v, :, pl.ds(ib0, _LANES)], bi.at[s], sem)

        def drain(bu, bi, sem):
            for s in range(_NS):
                pltpu.make_async_copy(
                    ut_h.at[dix_v, :, pl.ds(0, _LANES)], bu.at[s], sem).wait()
                pltpu.make_async_copy(
                    it_h.at[dix_v, :, pl.ds(0, _LANES)], bi.at[s], sem).wait()

        def extract(uv, iv, k0, loc0, bu, bi, ns=_NS):
            for s in range(ns):
                sfull = jnp.full((L,), s, jnp.int32)
                m = loc0 + s
                ulane = jnp.full((L,), uv[k0 + s] & (_LANES - 1), jnp.int32)
                urow_v[m, pl.ds(0, L)] = plsc.load_gather(
                    bu, [sfull, db_lo, di_sub, ulane])
                urow_v[m, pl.ds(L, L)] = plsc.load_gather(
                    bu, [sfull, db_hi, di_sub, ulane])
                ilane = jnp.full((L,), iv[k0 + s] & (_LANES - 1), jnp.int32)
                irow_v[m, pl.ds(0, L)] = plsc.load_gather(
                    bi, [sfull, db_lo, di_sub, ilane])
                irow_v[m, pl.ds(L, L)] = plsc.load_gather(
                    bi, [sfull, db_hi, di_sub, ilane])

        for q in range(4):
            h0 = q * QTR
            uv0 = uid_v[pl.ds(h0, L)]
            iv0 = iid_v[pl.ds(h0, L)]
            fire(uv0, iv0, 0, ua_v, ia_v, semA)

            def pipe_body(j, carry, h0=h0):
                loc0 = 2 * _NS * j
                n0 = h0 + loc0
                uv = uid_v[pl.ds(n0, L)]
                iv = iid_v[pl.ds(n0, L)]
                fire(uv, iv, _NS, ub_v, ib_v, semB)
                drain(ua_v, ia_v, semA)
                extract(uv, iv, 0, loc0, ua_v, ia_v)
                fire(uv, iv, 2 * _NS, ua_v, ia_v, semA)
                drain(ub_v, ib_v, semB)
                extract(uv, iv, _NS, loc0 + _NS, ub_v, ib_v)
                return carry

            lax.fori_loop(0, n_pipe, pipe_body, 0)

            loc0 = 2 * _NS * n_pipe
            uvt = uid_v[pl.ds(h0 + loc0, L)]
            ivt = iid_v[pl.ds(h0 + loc0, L)]
            drain(ua_v, ia_v, semA)
            extract(uvt, ivt, 0, loc0, ua_v, ia_v, ns=tail)

            def dot_body(g, carry, h0=h0):
                rows = riota + g * L
                acc = jnp.zeros((L,), jnp.float32)
                for d in range(D):
                    dcol = jnp.full((L,), d, jnp.int32)
                    uc = plsc.load_gather(urow_v, [rows, dcol])
                    ic = plsc.load_gather(irow_v, [rows, dcol])
                    acc = acc + uc * ic
                sig = 1.0 / (1.0 + jnp.exp(-acc))
                dv = diff_v[pl.ds(h0 + g * L, L)]
                y = sig * dv * w + b
                out_v[pl.ds(h0 + g * L, L)] = 1.0 / (1.0 + jnp.exp(-y))
                return carry

            lax.fori_loop(0, n_grp, dot_body, 0)

        pltpu.sync_copy(out_v, out_h.at[pl.ds(base, b_per_w)])

    return mf_kernel(user_id, item_id, diff, utT, itT, w16, b16, dblk_ix)


# speedup vs baseline: 3.2515x; 1.0154x over previous
import functools

import jax
import jax.numpy as jnp
from jax import lax
from jax.experimental import pallas as pl
from jax.experimental.pallas import tpu as pltpu
from jax.experimental.pallas import tpu_sc as plsc

_LANES = 128
_NS = 3


def kernel(user_id, item_id, diff, user_table, item_table, W_out, b_out):
    B = user_id.shape[0]
    D = user_table.shape[1]
    V = user_table.shape[0]
    info = plsc.get_sparse_core_info()
    NC, NS, L = info.num_cores, info.num_subcores, info.num_lanes
    NW = NC * NS
    b_per_w = B // NW
    QTR = b_per_w // 4
    n_grp = QTR // L
    n_pipe = QTR // (2 * _NS)
    tail = QTR - 2 * _NS * n_pipe
    NDB = D // 8

    utT = user_table.T.reshape(NDB, 8, V)
    itT = item_table.T.reshape(NDB, 8, V)
    dblk_ix = jnp.arange(NDB, dtype=jnp.int32)
    w16 = jnp.broadcast_to(W_out.reshape(1), (L,))
    b16 = jnp.broadcast_to(b_out, (L,))

    mesh = plsc.VectorSubcoreMesh(core_axis_name="c", subcore_axis_name="s")

    @functools.partial(
        pl.kernel,
        mesh=mesh,
        out_type=jax.ShapeDtypeStruct((B,), jnp.float32),
        scratch_types=[
            pltpu.VMEM((b_per_w + L,), jnp.int32),
            pltpu.VMEM((b_per_w + L,), jnp.int32),
            pltpu.VMEM((NDB,), jnp.int32),
            pltpu.VMEM((_NS, NDB, 8, _LANES), jnp.float32),
            pltpu.VMEM((_NS, NDB, 8, _LANES), jnp.float32),
            pltpu.VMEM((_NS, NDB, 8, _LANES), jnp.float32),
            pltpu.VMEM((_NS, NDB, 8, _LANES), jnp.float32),
            pltpu.VMEM((QTR, D), jnp.float32),
            pltpu.VMEM((QTR, D), jnp.float32),
            pltpu.VMEM((b_per_w,), jnp.float32),
            pltpu.VMEM((b_per_w,), jnp.float32),
            pltpu.VMEM((L,), jnp.float32),
            pltpu.VMEM((L,), jnp.float32),
            pltpu.SemaphoreType.DMA,
            pltpu.SemaphoreType.DMA,
        ],
        compiler_params=pltpu.CompilerParams(needs_layout_passes=False),
    )
    def mf_kernel(uid_h, iid_h, diff_h, ut_h, it_h, w_h, b_h, dix_h, out_h,
                  uid_v, iid_v, dix_v, ua_v, ia_v, ub_v, ib_v, urow_v,
                  irow_v, diff_v, out_v, w_v, b_v, semA, semB):
        wid = lax.axis_index("s") * NC + lax.axis_index("c")
        base = wid * b_per_w

        pltpu.sync_copy(uid_h.at[pl.ds(base, b_per_w)],
                        uid_v.at[pl.ds(0, b_per_w)])
        pltpu.sync_copy(iid_h.at[pl.ds(base, b_per_w)],
                        iid_v.at[pl.ds(0, b_per_w)])
        pltpu.sync_copy(diff_h.at[pl.ds(base, b_per_w)], diff_v)
        pltpu.sync_copy(w_h, w_v)
        pltpu.sync_copy(b_h, b_v)
        pltpu.sync_copy(dix_h, dix_v)

        zpad = jnp.zeros((L,), jnp.int32)
        uid_v[pl.ds(b_per_w, L)] = zpad
        iid_v[pl.ds(b_per_w, L)] = zpad

        w = w_v[...]
        b = b_v[...]
        riota = lax.iota(jnp.int32, L)
        diota = lax.iota(jnp.int32, L)
        db_lo = lax.shift_right_logical(diota, 3)
        db_hi = lax.shift_right_logical(diota + L, 3)
        di_sub = diota & 7

        def fire(uv, iv, k0, bu, bi, sem):
            for s in range(_NS):
                uid = uv[k0 + s]
                ub0 = pl.multiple_of(uid & ~(_LANES - 1), _LANES)
                pltpu.async_copy(
                    ut_h.at[dix_v, :, pl.ds(ub0, _LANES)], bu.at[s], sem)
                iid = iv[k0 + s]
                ib0 = pl.multiple_of(iid & ~(_LANES - 1), _LANES)
                pltpu.async_copy(
                    it_h.at[dix_v, :, pl.ds(ib0, _LANES)], bi.at[s], sem)

        def drain(bu, bi, sem):
            for s in range(_NS):
                pltpu.make_async_copy(
                    ut_h.at[dix_v, :, pl.ds(0, _LANES)], bu.at[s], sem).wait()
                pltpu.make_async_copy(
                    it_h.at[dix_v, :, pl.ds(0, _LANES)], bi.at[s], sem).wait()

        def extract(uv, iv, k0, loc0, bu, bi, ns=_NS):
            for s in range(ns):
                sfull = jnp.full((L,), s, jnp.int32)
                m = loc0 + s
                ulane = jnp.full((L,), uv[k0 + s] & (_LANES - 1), jnp.int32)
                urow_v[m, pl.ds(0, L)] = plsc.load_gather(
                    bu, [sfull, db_lo, di_sub, ulane])
                urow_v[m, pl.ds(L, L)] = plsc.load_gather(
                    bu, [sfull, db_hi, di_sub, ulane])
                ilane = jnp.full((L,), iv[k0 + s] & (_LANES - 1), jnp.int32)
                irow_v[m, pl.ds(0, L)] = plsc.load_gather(
                    bi, [sfull, db_lo, di_sub, ilane])
                irow_v[m, pl.ds(L, L)] = plsc.load_gather(
                    bi, [sfull, db_hi, di_sub, ilane])

        uv0 = uid_v[pl.ds(0, L)]
        iv0 = iid_v[pl.ds(0, L)]
        fire(uv0, iv0, 0, ua_v, ia_v, semA)

        for q in range(4):
            h0 = q * QTR

            def pipe_body(j, carry, h0=h0):
                loc0 = 2 * _NS * j
                n0 = h0 + loc0
                uv = uid_v[pl.ds(n0, L)]
                iv = iid_v[pl.ds(n0, L)]
                fire(uv, iv, _NS, ub_v, ib_v, semB)
                drain(ua_v, ia_v, semA)
                extract(uv, iv, 0, loc0, ua_v, ia_v)
                fire(uv, iv, 2 * _NS, ua_v, ia_v, semA)
                drain(ub_v, ib_v, semB)
                extract(uv, iv, _NS, loc0 + _NS, ub_v, ib_v)
                return carry

            lax.fori_loop(0, n_pipe, pipe_body, 0)

            loc0 = 2 * _NS * n_pipe
            uvt = uid_v[pl.ds(h0 + loc0, L)]
            ivt = iid_v[pl.ds(h0 + loc0, L)]
            drain(ua_v, ia_v, semA)
            extract(uvt, ivt, 0, loc0, ua_v, ia_v, ns=tail)

            if q < 3:
                uvn = uid_v[pl.ds((q + 1) * QTR, L)]
                ivn = iid_v[pl.ds((q + 1) * QTR, L)]
                fire(uvn, ivn, 0, ua_v, ia_v, semA)

            def dot_body(g, carry, h0=h0):
                rows = riota + g * L
                acc = jnp.zeros((L,), jnp.float32)
                for d in range(D):
                    dcol = jnp.full((L,), d, jnp.int32)
                    uc = plsc.load_gather(urow_v, [rows, dcol])
                    ic = plsc.load_gather(irow_v, [rows, dcol])
                    acc = acc + uc * ic
                sig = 1.0 / (1.0 + jnp.exp(-acc))
                dv = diff_v[pl.ds(h0 + g * L, L)]
                y = sig * dv * w + b
                out_v[pl.ds(h0 + g * L, L)] = 1.0 / (1.0 + jnp.exp(-y))
                return carry

            lax.fori_loop(0, n_grp, dot_body, 0)

        pltpu.sync_copy(out_v, out_h.at[pl.ds(base, b_per_w)])

    return mf_kernel(user_id, item_id, diff, utT, itT, w16, b16, dblk_ix)


# speedup vs baseline: 4.0025x; 1.2310x over previous
import functools

import jax
import jax.numpy as jnp
from jax import lax
from jax.experimental import pallas as pl
from jax.experimental.pallas import tpu as pltpu
from jax.experimental.pallas import tpu_sc as plsc

_LANES = 128
_NS = 3
_PASS = 128
_TCN = 64


def _sc_part(user_id, item_id, diff, utT, itT, w16, b16, dblk_ix):
    B = user_id.shape[0]
    NDB, _, V = utT.shape
    D = NDB * 8
    info = plsc.get_sparse_core_info()
    NC, NS, L = info.num_cores, info.num_subcores, info.num_lanes
    NW = NC * NS
    b_per_w = B // NW
    n_pass = b_per_w // _PASS
    n_grp = _PASS // L
    n_pipe = _PASS // (2 * _NS)
    tail = _PASS - 2 * _NS * n_pipe

    mesh = plsc.VectorSubcoreMesh(core_axis_name="c", subcore_axis_name="s")

    @functools.partial(
        pl.kernel,
        mesh=mesh,
        out_type=jax.ShapeDtypeStruct((B,), jnp.float32),
        scratch_types=[
            pltpu.VMEM((b_per_w + L,), jnp.int32),
            pltpu.VMEM((b_per_w + L,), jnp.int32),
            pltpu.VMEM((NDB,), jnp.int32),
            pltpu.VMEM((_NS, NDB, 8, _LANES), jnp.float32),
            pltpu.VMEM((_NS, NDB, 8, _LANES), jnp.float32),
            pltpu.VMEM((_NS, NDB, 8, _LANES), jnp.float32),
            pltpu.VMEM((_NS, NDB, 8, _LANES), jnp.float32),
            pltpu.VMEM((_PASS, D), jnp.float32),
            pltpu.VMEM((_PASS, D), jnp.float32),
            pltpu.VMEM((b_per_w,), jnp.float32),
            pltpu.VMEM((b_per_w,), jnp.float32),
            pltpu.VMEM((L,), jnp.float32),
            pltpu.VMEM((L,), jnp.float32),
            pltpu.SemaphoreType.DMA,
            pltpu.SemaphoreType.DMA,
        ],
        compiler_params=pltpu.CompilerParams(needs_layout_passes=False),
    )
    def mf_kernel(uid_h, iid_h, diff_h, ut_h, it_h, w_h, b_h, dix_h, out_h,
                  uid_v, iid_v, dix_v, ua_v, ia_v, ub_v, ib_v, urow_v,
                  irow_v, diff_v, out_v, w_v, b_v, semA, semB):
        wid = lax.axis_index("s") * NC + lax.axis_index("c")
        base = wid * b_per_w

        pltpu.sync_copy(uid_h.at[pl.ds(base, b_per_w)],
                        uid_v.at[pl.ds(0, b_per_w)])
        pltpu.sync_copy(iid_h.at[pl.ds(base, b_per_w)],
                        iid_v.at[pl.ds(0, b_per_w)])
        pltpu.sync_copy(diff_h.at[pl.ds(base, b_per_w)], diff_v)
        pltpu.sync_copy(w_h, w_v)
        pltpu.sync_copy(b_h, b_v)
        pltpu.sync_copy(dix_h, dix_v)

        zpad = jnp.zeros((L,), jnp.int32)
        uid_v[pl.ds(b_per_w, L)] = zpad
        iid_v[pl.ds(b_per_w, L)] = zpad

        w = w_v[...]
        b = b_v[...]
        riota = lax.iota(jnp.int32, L)
        diota = lax.iota(jnp.int32, L)
        db_lo = lax.shift_right_logical(diota, 3)
        db_hi = lax.shift_right_logical(diota + L, 3)
        di_sub = diota & 7

        def fire(uv, iv, k0, bu, bi, sem):
            for s in range(_NS):
                uid = uv[k0 + s]
                ub0 = pl.multiple_of(uid & ~(_LANES - 1), _LANES)
                pltpu.async_copy(
                    ut_h.at[dix_v, :, pl.ds(ub0, _LANES)], bu.at[s], sem)
                iid = iv[k0 + s]
                ib0 = pl.multiple_of(iid & ~(_LANES - 1), _LANES)
                pltpu.async_copy(
                    it_h.at[dix_v, :, pl.ds(ib0, _LANES)], bi.at[s], sem)

        def drain(bu, bi, sem):
            for s in range(_NS):
                pltpu.make_async_copy(
                    ut_h.at[dix_v, :, pl.ds(0, _LANES)], bu.at[s], sem).wait()
                pltpu.make_async_copy(
                    it_h.at[dix_v, :, pl.ds(0, _LANES)], bi.at[s], sem).wait()

        def extract(uv, iv, k0, loc0, bu, bi, ns=_NS):
            for s in range(ns):
                sfull = jnp.full((L,), s, jnp.int32)
                m = loc0 + s
                ulane = jnp.full((L,), uv[k0 + s] & (_LANES - 1), jnp.int32)
                urow_v[m, pl.ds(0, L)] = plsc.load_gather(
                    bu, [sfull, db_lo, di_sub, ulane])
                urow_v[m, pl.ds(L, L)] = plsc.load_gather(
                    bu, [sfull, db_hi, di_sub, ulane])
                ilane = jnp.full((L,), iv[k0 + s] & (_LANES - 1), jnp.int32)
                irow_v[m, pl.ds(0, L)] = plsc.load_gather(
                    bi, [sfull, db_lo, di_sub, ilane])
                irow_v[m, pl.ds(L, L)] = plsc.load_gather(
                    bi, [sfull, db_hi, di_sub, ilane])

        uv0 = uid_v[pl.ds(0, L)]
        iv0 = iid_v[pl.ds(0, L)]
        fire(uv0, iv0, 0, ua_v, ia_v, semA)

        for q in range(n_pass):
            h0 = q * _PASS

            def pipe_body(j, carry, h0=h0):
                loc0 = 2 * _NS * j
                n0 = h0 + loc0
                uv = uid_v[pl.ds(n0, L)]
                iv = iid_v[pl.ds(n0, L)]
                fire(uv, iv, _NS, ub_v, ib_v, semB)
                drain(ua_v, ia_v, semA)
                extract(uv, iv, 0, loc0, ua_v, ia_v)
                fire(uv, iv, 2 * _NS, ua_v, ia_v, semA)
                drain(ub_v, ib_v, semB)
                extract(uv, iv, _NS, loc0 + _NS, ub_v, ib_v)
                return carry

            lax.fori_loop(0, n_pipe, pipe_body, 0)

            loc0 = 2 * _NS * n_pipe
            uvt = uid_v[pl.ds(h0 + loc0, L)]
            ivt = iid_v[pl.ds(h0 + loc0, L)]
            drain(ua_v, ia_v, semA)
            extract(uvt, ivt, 0, loc0, ua_v, ia_v, ns=tail)

            if q < n_pass - 1:
                uvn = uid_v[pl.ds((q + 1) * _PASS, L)]
                ivn = iid_v[pl.ds((q + 1) * _PASS, L)]
                fire(uvn, ivn, 0, ua_v, ia_v, semA)

            def dot_body(g, carry, h0=h0):
                rows = riota + g * L
                acc = jnp.zeros((L,), jnp.float32)
                for d in range(D):
                    dcol = jnp.full((L,), d, jnp.int32)
                    uc = plsc.load_gather(urow_v, [rows, dcol])
                    ic = plsc.load_gather(irow_v, [rows, dcol])
                    acc = acc + uc * ic
                sig = 1.0 / (1.0 + jnp.exp(-acc))
                dv = diff_v[pl.ds(h0 + g * L, L)]
                y = sig * dv * w + b
                out_v[pl.ds(h0 + g * L, L)] = 1.0 / (1.0 + jnp.exp(-y))
                return carry

            lax.fori_loop(0, n_grp, dot_body, 0)

        pltpu.sync_copy(out_v, out_h.at[pl.ds(base, b_per_w)])

    return mf_kernel(user_id, item_id, diff, utT, itT, w16, b16, dblk_ix)


def _tc_part(user_id, item_id, diff, utT2, itT2, W_out, b_out):
    B = user_id.shape[0]
    D = utT2.shape[0]
    steps = B // _TCN
    uoh = jax.nn.one_hot(user_id & (_LANES - 1), _LANES, dtype=jnp.float32)
    ioh = jax.nn.one_hot(item_id & (_LANES - 1), _LANES, dtype=jnp.float32)
    diff2 = diff.reshape(steps, _TCN)
    w11 = W_out.reshape(1, 1)
    b11 = b_out.reshape(1, 1)

    grid_spec = pltpu.PrefetchScalarGridSpec(
        num_scalar_prefetch=2,
        grid=(steps,),
        in_specs=[
            pl.BlockSpec(memory_space=pltpu.HBM),
            pl.BlockSpec(memory_space=pltpu.HBM),
            pl.BlockSpec((_TCN, _LANES), lambda s, *_: (s, 0)),
            pl.BlockSpec((_TCN, _LANES), lambda s, *_: (s, 0)),
            pl.BlockSpec((steps, _TCN), lambda s, *_: (0, 0)),
            pl.BlockSpec((1, 1), lambda s, *_: (0, 0)),
            pl.BlockSpec((1, 1), lambda s, *_: (0, 0)),
        ],
        out_specs=pl.BlockSpec((steps, _TCN), lambda s, *_: (0, 0)),
        scratch_shapes=[
            pltpu.VMEM((2, _TCN, D, _LANES), jnp.float32),
            pltpu.VMEM((2, _TCN, D, _LANES), jnp.float32),
            pltpu.VMEM((D, _TCN), jnp.float32),
            pltpu.VMEM((D, _TCN), jnp.float32),
            pltpu.SemaphoreType.DMA((2,)),
        ],
    )

    def body(su, si, ut_ref, it_ref, uoh_ref, ioh_ref, dv_ref, w_ref, b_ref,
             out_ref, ub_v, ib_v, ux_v, ix_v, sems):
        s = pl.program_id(0)

        def fire(step, slot):
            for k in range(_TCN):
                u = su[step * _TCN + k]
                u0 = pl.multiple_of((u >> 7) * _LANES, _LANES)
                pltpu.make_async_copy(
                    ut_ref.at[:, pl.ds(u0, _LANES)],
                    ub_v.at[slot, k], sems.at[slot]).start()
                i = si[step * _TCN + k]
                i0 = pl.multiple_of((i >> 7) * _LANES, _LANES)
                pltpu.make_async_copy(
                    it_ref.at[:, pl.ds(i0, _LANES)],
                    ib_v.at[slot, k], sems.at[slot]).start()

        @pl.when(s == 0)
        def _():
            fire(0, 0)

        @pl.when(s < steps - 1)
        def _():
            fire(s + 1, (s + 1) % 2)

        slot = s % 2
        for k in range(_TCN):
            pltpu.make_async_copy(
                ut_ref.at[:, pl.ds(0, _LANES)],
                ub_v.at[slot, k], sems.at[slot]).wait()
            pltpu.make_async_copy(
                it_ref.at[:, pl.ds(0, _LANES)],
                ib_v.at[slot, k], sems.at[slot]).wait()

        for k in range(_TCN):
            um = uoh_ref[k, :].reshape(1, _LANES)
            uc = jnp.sum(ub_v[slot, k] * um, axis=1, keepdims=True)
            ux_v[:, pl.ds(k, 1)] = uc
            im = ioh_ref[k, :].reshape(1, _LANES)
            ic = jnp.sum(ib_v[slot, k] * im, axis=1, keepdims=True)
            ix_v[:, pl.ds(k, 1)] = ic

        scores = jnp.sum(ux_v[...] * ix_v[...], axis=0, keepdims=True)
        sig = 1.0 / (1.0 + jnp.exp(-scores))
        dv = dv_ref[pl.ds(s, 1), :]
        y = sig * dv * w_ref[0, 0] + b_ref[0, 0]
        out_ref[pl.ds(s, 1), :] = 1.0 / (1.0 + jnp.exp(-y))

    out2 = pl.pallas_call(
        body,
        grid_spec=grid_spec,
        out_shape=jax.ShapeDtypeStruct((steps, _TCN), jnp.float32),
    )(user_id, item_id, utT2, itT2, uoh, ioh, diff2, w11, b11)
    return out2.reshape(B)


def kernel(user_id, item_id, diff, user_table, item_table, W_out, b_out):
    B = user_id.shape[0]
    D = user_table.shape[1]
    V = user_table.shape[0]
    NDB = D // 8
    Bsc = B // 2

    utT3 = user_table.T.reshape(NDB, 8, V)
    itT3 = item_table.T.reshape(NDB, 8, V)
    utT2 = user_table.T
    itT2 = item_table.T
    dblk_ix = jnp.arange(NDB, dtype=jnp.int32)
    w16 = jnp.broadcast_to(W_out.reshape(1), (16,))
    b16 = jnp.broadcast_to(b_out, (16,))

    out_sc = _sc_part(user_id[:Bsc], item_id[:Bsc], diff[:Bsc],
                      utT3, itT3, w16, b16, dblk_ix)
    out_tc = _tc_part(user_id[Bsc:], item_id[Bsc:], diff[Bsc:],
                      utT2, itT2, W_out, b_out)
    return jnp.concatenate([out_sc, out_tc])
